# baseline (speedup 1.0000x reference)
import jax
import jax.numpy as jnp
from jax.experimental import pallas as pl
from jax.experimental.pallas import tpu as pltpu

_BLOCK_N = 20000
_NSEM = 5


def _body(w_ref, b_ref, x0_ref, out_hbm, scratch, sems):
    s = w_ref[4] + w_ref[5] + w_ref[6] + w_ref[7] + w_ref[8]
    row = x0_ref[0:1, :] * s + b_ref[0]
    scratch[...] = jnp.broadcast_to(row, scratch.shape)

    nchunks = out_hbm.shape[0] // _BLOCK_N

    def copy(i):
        return pltpu.make_async_copy(
            scratch,
            out_hbm.at[pl.ds(i * _BLOCK_N, _BLOCK_N), :],
            sems.at[i % _NSEM],
        )

    for i in range(nchunks):
        if i >= _NSEM:
            copy(i - _NSEM).wait()
        copy(i).start()
    for i in range(max(nchunks - _NSEM, 0), nchunks):
        copy(i).wait()


def kernel(X, c, conv_w, conv_b):
    N, C = X.shape
    return pl.pallas_call(
        _body,
        grid=(1,),
        in_specs=[
            pl.BlockSpec(memory_space=pltpu.SMEM),
            pl.BlockSpec(memory_space=pltpu.SMEM),
            pl.BlockSpec((8, C), lambda i: (0, 0)),
        ],
        out_specs=pl.BlockSpec(memory_space=pl.ANY),
        out_shape=jax.ShapeDtypeStruct((N, C), X.dtype),
        scratch_shapes=[
            pltpu.VMEM((_BLOCK_N, C), jnp.float32),
            pltpu.SemaphoreType.DMA((_NSEM,)),
        ],
    )(conv_w.reshape(9), conv_b, X)


# speedup vs baseline: 1.0197x; 1.0197x over previous
import jax
import jax.numpy as jnp
from jax.experimental import pallas as pl
from jax.experimental.pallas import tpu as pltpu

_BLOCK_N = 2000
_NSEM = 16


def _body(w_ref, b_ref, x0_ref, out_hbm, scratch, sems):
    s = w_ref[4] + w_ref[5] + w_ref[6] + w_ref[7] + w_ref[8]
    row = x0_ref[0:1, :] * s + b_ref[0]
    scratch[...] = jnp.broadcast_to(row, scratch.shape)

    nchunks = out_hbm.shape[0] // _BLOCK_N

    def copy(i):
        return pltpu.make_async_copy(
            scratch,
            out_hbm.at[pl.ds(i * _BLOCK_N, _BLOCK_N), :],
            sems.at[i % _NSEM],
        )

    for i in range(nchunks):
        if i >= _NSEM:
            copy(i - _NSEM).wait()
        copy(i).start()
    for i in range(max(nchunks - _NSEM, 0), nchunks):
        copy(i).wait()


def kernel(X, c, conv_w, conv_b):
    N, C = X.shape
    return pl.pallas_call(
        _body,
        grid=(1,),
        in_specs=[
            pl.BlockSpec(memory_space=pltpu.SMEM),
            pl.BlockSpec(memory_space=pltpu.SMEM),
            pl.BlockSpec((8, C), lambda i: (0, 0)),
        ],
        out_specs=pl.BlockSpec(memory_space=pl.ANY),
        out_shape=jax.ShapeDtypeStruct((N, C), X.dtype),
        scratch_shapes=[
            pltpu.VMEM((_BLOCK_N, C), jnp.float32),
            pltpu.SemaphoreType.DMA((_NSEM,)),
        ],
    )(conv_w.reshape(9), conv_b, X)


# speedup vs baseline: 1.0654x; 1.0448x over previous
import jax
import jax.numpy as jnp
from jax.experimental import pallas as pl
from jax.experimental.pallas import tpu as pltpu

_BLOCK_N = 4000


def _body(w_ref, b_ref, x0_ref, out_ref):
    s = w_ref[4] + w_ref[5] + w_ref[6] + w_ref[7] + w_ref[8]
    row = x0_ref[0:1, :] * s + b_ref[0]
    out_ref[...] = jnp.broadcast_to(row, out_ref.shape)


def kernel(X, c, conv_w, conv_b):
    N, C = X.shape
    grid = (N // _BLOCK_N,)
    return pl.pallas_call(
        _body,
        grid=grid,
        in_specs=[
            pl.BlockSpec(memory_space=pltpu.SMEM),
            pl.BlockSpec(memory_space=pltpu.SMEM),
            pl.BlockSpec((8, C), lambda i: (0, 0)),
        ],
        out_specs=pl.BlockSpec((_BLOCK_N, C), lambda i: (i, 0)),
        out_shape=jax.ShapeDtypeStruct((N, C), X.dtype),
    )(conv_w.reshape(9), conv_b, X)
